# baseline (speedup 1.0000x reference)
import numpy as np

import jax
import jax.numpy as jnp
from jax import lax
from jax.experimental import pallas as pl
from jax.experimental.pallas import tpu as pltpu
from jax.experimental.pallas import tpu_sc as plsc

_B, _D = 64, 8192
_K = 1639
_L = 16
_NV = _D // _L
_U = 8
_NC, _NS = 2, 16
_NW = _NC * _NS
_RPW = _B // _NW
_IMIN = np.int32(-2147483648)
_LO = np.float32(0.76)
_HI = np.float32(0.92)
_NEG_INF = np.float32(-np.inf)
_POS_INF = np.float32(np.inf)


def _keys(x):
    b = lax.bitcast_convert_type(x, jnp.int32)
    return jnp.where(b >= 0, b ^ _IMIN, ~b)


def _find_digit(hist_v, kprime, iota):
    S = hist_v[pl.ds(0, _L)]
    for j in range(1, 16):
        S = S + hist_v[pl.ds(j * _L, _L)]
    sh = jnp.flip(jnp.cumsum(jnp.flip(S)))
    cnt1 = jnp.sum((sh >= kprime).astype(jnp.int32))
    hstar = cnt1 - 1
    hab = jnp.sum(jnp.where(iota == cnt1, sh, 0))
    w = plsc.load_gather(hist_v, [iota * _L + hstar])
    sw = jnp.flip(jnp.cumsum(jnp.flip(w))) + hab
    cnt2 = jnp.sum((sw >= kprime).astype(jnp.int32))
    gtv = jnp.sum(jnp.where(iota == cnt2, sw, 0))
    gt = jnp.where(cnt2 == _L, hab, gtv)
    dstar = hstar * _L + cnt2 - 1
    return dstar, kprime - gt


def _compact(x_v, cand_v, lo_f, hi_f, zvec, waits=None):
    def body(i, c):
        off, nhi_v = c
        x = x_v[pl.ds(i * _L, _L)]
        mhi = x >= hi_f
        match = (x >= lo_f) & jnp.logical_not(mhi)
        idx = off + jnp.cumsum(match.astype(jnp.int32)) - 1
        plsc.store_scatter(cand_v, [idx], x, mask=match)
        return (off + plsc.all_reduce_population_count(match),
                nhi_v + mhi.astype(jnp.int32))

    carry = (zvec, zvec)
    if waits is None:
        carry = plsc.parallel_loop(0, _NV, 1, unroll=_U, carry=carry)(body)
    else:
        nchunk = len(waits)
        per = _NV // nchunk
        for c in range(nchunk):
            waits[c]()
            carry = plsc.parallel_loop(c * per, (c + 1) * per, 1,
                                       unroll=_U, carry=carry)(body)
    off, nhi_v = carry
    return jnp.max(off), jnp.sum(nhi_v)


def _select_row(x_v, cand_v, hist_v, iota, ones, zeros, waits=None):
    k = jnp.int32(_K)
    m, nhi = _compact(x_v, cand_v, _LO, _HI, zeros, waits)
    good = (nhi < k) & (k <= nhi + m)
    hi_side = nhi >= k
    lo_f = jnp.where(good, _LO, jnp.where(hi_side, _HI, _NEG_INF))
    hi_f = jnp.where(good, _HI, jnp.where(hi_side, _POS_INF, _LO))
    rank = jnp.where(good, k - nhi, jnp.where(hi_side, k, k - (nhi + m)))
    m_fin = jnp.where(good, m,
                      jnp.where(hi_side, nhi, jnp.int32(_D) - nhi - m))

    @pl.when(jnp.logical_not(good))
    def _():
        _compact(x_v, cand_v, lo_f, hi_f, zeros)

    plsc.store_scatter(cand_v, [jnp.broadcast_to(m_fin, (_L,)) + iota],
                       jnp.broadcast_to(hi_f, (_L,)))
    nblk = m_fin // _L + 1

    kprime = rank
    prefix = jnp.int32(0)
    pmask = jnp.int32(0)
    for p in range(4):
        shift = 24 - 8 * p
        for j in range(16):
            hist_v[pl.ds(j * _L, _L)] = zeros

        @plsc.parallel_loop(0, nblk, 1, unroll=2)
        def _(i, prefix=prefix, pmask=pmask, shift=shift, lo_f=lo_f,
              hi_f=hi_f):
            xc = cand_v[pl.ds(i * _L, _L)]
            ku = _keys(xc)
            inr = (xc >= lo_f) & jnp.logical_not(xc >= hi_f)
            match = inr & ((ku & pmask) == prefix)
            lo_nib = lax.shift_right_logical(ku, shift) & 0xF
            hi_nib = lax.shift_right_logical(ku, shift + 4) & 0xF
            tidx = lax.shift_left(lo_nib, 4) | hi_nib
            plsc.addupdate_scatter(hist_v, [tidx], ones, mask=match)

        d, kprime = _find_digit(hist_v, kprime, iota)
        prefix = prefix | lax.shift_left(d, jnp.int32(shift))
        pmask = pmask | lax.shift_left(jnp.int32(0xFF), jnp.int32(shift))

    bits = jnp.where(prefix < 0, prefix ^ _IMIN, ~prefix)
    return plsc.bitcast(jnp.broadcast_to(bits, (_L,)), jnp.float32)


def _mask_row(x_v, vstar_v):
    @plsc.parallel_loop(0, _NV, 1, unroll=_U)
    def _(i):
        x = x_v[pl.ds(i * _L, _L)]
        x_v[pl.ds(i * _L, _L)] = jnp.where(x >= vstar_v, x, 0.0)


def _kwta_body(x_hbm, o_hbm, x0_v, x1_v, cand_v, hist_v, sem0, sem1, semo):
    wid = lax.axis_index("s") * _NC + lax.axis_index("c")
    iota = lax.iota(jnp.int32, _L)
    ones = jnp.ones((_L,), jnp.int32)
    zeros = jnp.zeros((_L,), jnp.int32)
    row0 = wid * _RPW
    bufs = [x0_v, x1_v]
    isems = [sem0, sem1]
    nchunk = 4
    per_el = _D // nchunk
    chunk_copies = [
        pltpu.async_copy(x_hbm.at[row0, pl.ds(c * per_el, per_el)],
                         x0_v.at[pl.ds(c * per_el, per_el)], sem0)
        for c in range(nchunk)]
    incopy = [None,
              pltpu.async_copy(x_hbm.at[row0 + 1], x1_v, isems[1])
              if _RPW > 1 else None]
    ocopy = [None, None]
    for r in range(_RPW):
        b = r % 2
        waits = None
        if r == 0:
            waits = [cc.wait for cc in chunk_copies]
        else:
            incopy[b].wait()
        if ocopy[b] is not None:
            ocopy[b].wait()
        vstar = _select_row(bufs[b], cand_v, hist_v, iota, ones, zeros,
                            waits)
        _mask_row(bufs[b], vstar)
        ocopy[b] = pltpu.async_copy(bufs[b], o_hbm.at[row0 + r], semo)
        if r + 2 < _RPW:
            ocopy[b].wait()
            ocopy[b] = None
            incopy[b] = pltpu.async_copy(x_hbm.at[row0 + r + 2], bufs[b],
                                         isems[b])
    for b in range(min(2, _RPW)):
        if ocopy[b] is not None:
            ocopy[b].wait()


def kernel(inputs):
    mesh = plsc.VectorSubcoreMesh(
        core_axis_name="c", subcore_axis_name="s", num_cores=_NC,
        num_subcores=_NS)
    f = pl.kernel(
        _kwta_body,
        out_type=jax.ShapeDtypeStruct((_B, _D), jnp.float32),
        mesh=mesh,
        scratch_types=[
            pltpu.VMEM((_D,), jnp.float32),
            pltpu.VMEM((_D,), jnp.float32),
            pltpu.VMEM((_D + _L,), jnp.float32),
            pltpu.VMEM((256,), jnp.int32),
            pltpu.SemaphoreType.DMA,
            pltpu.SemaphoreType.DMA,
            pltpu.SemaphoreType.DMA,
        ],
        compiler_params=pltpu.CompilerParams(needs_layout_passes=False),
    )
    return f(inputs)


# speedup vs baseline: 1.0635x; 1.0635x over previous
import numpy as np

import jax
import jax.numpy as jnp
from jax import lax
from jax.experimental import pallas as pl
from jax.experimental.pallas import tpu as pltpu
from jax.experimental.pallas import tpu_sc as plsc

_B, _D = 64, 8192
_K = 1639
_L = 16
_NV = _D // _L
_U = 8
_NC, _NS = 2, 16
_NW = _NC * _NS
_RPW = _B // _NW
_IMIN = np.int32(-2147483648)
_LO = np.float32(0.76)
_HI = np.float32(0.92)
_NEG_INF = np.float32(-np.inf)
_POS_INF = np.float32(np.inf)


def _keys(x):
    b = lax.bitcast_convert_type(x, jnp.int32)
    return jnp.where(b >= 0, b ^ _IMIN, ~b)


def _find_digit(hist_v, kprime, iota):
    S = hist_v[pl.ds(0, _L)]
    for j in range(1, 16):
        S = S + hist_v[pl.ds(j * _L, _L)]
    sh = jnp.flip(jnp.cumsum(jnp.flip(S)))
    cnt1 = jnp.sum((sh >= kprime).astype(jnp.int32))
    hstar = cnt1 - 1
    hab = jnp.sum(jnp.where(iota == cnt1, sh, 0))
    w = plsc.load_gather(hist_v, [iota * _L + hstar])
    sw = jnp.flip(jnp.cumsum(jnp.flip(w))) + hab
    cnt2 = jnp.sum((sw >= kprime).astype(jnp.int32))
    gtv = jnp.sum(jnp.where(iota == cnt2, sw, 0))
    gt = jnp.where(cnt2 == _L, hab, gtv)
    dstar = hstar * _L + cnt2 - 1
    return dstar, kprime - gt


def _compact(x_v, cand_v, lo_f, hi_f, zvec):
    def body(i, c):
        off, nhi_v = c
        x = x_v[pl.ds(i * _L, _L)]
        mhi = x >= hi_f
        match = (x >= lo_f) & jnp.logical_not(mhi)
        idx = off + jnp.cumsum(match.astype(jnp.int32)) - 1
        plsc.store_scatter(cand_v, [idx], x, mask=match)
        return (off + plsc.all_reduce_population_count(match),
                nhi_v + mhi.astype(jnp.int32))

    off, nhi_v = plsc.parallel_loop(0, _NV, 1, unroll=_U,
                                    carry=(zvec, zvec))(body)
    return jnp.max(off), jnp.sum(nhi_v)


def _select_row(x_v, cand_v, hist_v, iota, ones, zeros):
    k = jnp.int32(_K)
    m, nhi = _compact(x_v, cand_v, _LO, _HI, zeros)
    good = (nhi < k) & (k <= nhi + m)
    hi_side = nhi >= k
    lo_f = jnp.where(good, _LO, jnp.where(hi_side, _HI, _NEG_INF))
    hi_f = jnp.where(good, _HI, jnp.where(hi_side, _POS_INF, _LO))
    rank = jnp.where(good, k - nhi, jnp.where(hi_side, k, k - (nhi + m)))
    m_fin = jnp.where(good, m,
                      jnp.where(hi_side, nhi, jnp.int32(_D) - nhi - m))

    @pl.when(jnp.logical_not(good))
    def _():
        _compact(x_v, cand_v, lo_f, hi_f, zeros)

    plsc.store_scatter(cand_v, [jnp.broadcast_to(m_fin, (_L,)) + iota],
                       jnp.broadcast_to(hi_f, (_L,)))
    nblk = m_fin // _L + 1

    def radix_body(p, c):
        kprime, prefix, pmask = c
        shift = jnp.int32(24) - jnp.int32(8) * p
        for j in range(16):
            hist_v[pl.ds(j * _L, _L)] = zeros

        @plsc.parallel_loop(0, nblk, 1, unroll=2)
        def _(i):
            xc = cand_v[pl.ds(i * _L, _L)]
            ku = _keys(xc)
            inr = (xc >= lo_f) & jnp.logical_not(xc >= hi_f)
            match = inr & ((ku & pmask) == prefix)
            lo_nib = lax.shift_right_logical(ku, shift) & 0xF
            hi_nib = lax.shift_right_logical(ku, shift + 4) & 0xF
            tidx = lax.shift_left(lo_nib, 4) | hi_nib
            plsc.addupdate_scatter(hist_v, [tidx], ones, mask=match)

        d, kprime = _find_digit(hist_v, kprime, iota)
        prefix = prefix | lax.shift_left(d, shift)
        pmask = pmask | lax.shift_left(jnp.int32(0xFF), shift)
        return (kprime, prefix, pmask)

    kprime, prefix, pmask = lax.fori_loop(
        0, 4, radix_body, (rank, jnp.int32(0), jnp.int32(0)))

    bits = jnp.where(prefix < 0, prefix ^ _IMIN, ~prefix)
    return plsc.bitcast(jnp.broadcast_to(bits, (_L,)), jnp.float32)


def _mask_row(x_v, vstar_v):
    @plsc.parallel_loop(0, _NV, 1, unroll=_U)
    def _(i):
        x = x_v[pl.ds(i * _L, _L)]
        x_v[pl.ds(i * _L, _L)] = jnp.where(x >= vstar_v, x, 0.0)


def _kwta_body(x_hbm, o_hbm, x0_v, x1_v, cand_v, hist_v, sem0, sem1, semo):
    wid = lax.axis_index("s") * _NC + lax.axis_index("c")
    iota = lax.iota(jnp.int32, _L)
    ones = jnp.ones((_L,), jnp.int32)
    zeros = jnp.zeros((_L,), jnp.int32)
    row0 = wid * _RPW
    bufs = [x0_v, x1_v]
    isems = [sem0, sem1]
    incopy = [pltpu.async_copy(x_hbm.at[row0 + r], bufs[r % 2], isems[r % 2])
              for r in range(min(2, _RPW))]
    ocopy = [None, None]
    for r in range(_RPW):
        b = r % 2
        incopy[b].wait()
        if ocopy[b] is not None:
            ocopy[b].wait()
        vstar = _select_row(bufs[b], cand_v, hist_v, iota, ones, zeros)
        _mask_row(bufs[b], vstar)
        ocopy[b] = pltpu.async_copy(bufs[b], o_hbm.at[row0 + r], semo)
        if r + 2 < _RPW:
            ocopy[b].wait()
            ocopy[b] = None
            incopy[b] = pltpu.async_copy(x_hbm.at[row0 + r + 2], bufs[b],
                                         isems[b])
    for b in range(min(2, _RPW)):
        if ocopy[b] is not None:
            ocopy[b].wait()


def kernel(inputs):
    mesh = plsc.VectorSubcoreMesh(
        core_axis_name="c", subcore_axis_name="s", num_cores=_NC,
        num_subcores=_NS)
    f = pl.kernel(
        _kwta_body,
        out_type=jax.ShapeDtypeStruct((_B, _D), jnp.float32),
        mesh=mesh,
        scratch_types=[
            pltpu.VMEM((_D,), jnp.float32),
            pltpu.VMEM((_D,), jnp.float32),
            pltpu.VMEM((_D + _L,), jnp.float32),
            pltpu.VMEM((256,), jnp.int32),
            pltpu.SemaphoreType.DMA,
            pltpu.SemaphoreType.DMA,
            pltpu.SemaphoreType.DMA,
        ],
        compiler_params=pltpu.CompilerParams(needs_layout_passes=False),
    )
    return f(inputs)


# speedup vs baseline: 1.0844x; 1.0197x over previous
import numpy as np

import jax
import jax.numpy as jnp
from jax import lax
from jax.experimental import pallas as pl
from jax.experimental.pallas import tpu as pltpu
from jax.experimental.pallas import tpu_sc as plsc

_B, _D = 64, 8192
_K = 1639
_L = 16
_NV = _D // _L
_U = 8
_NC, _NS = 2, 16
_NW = _NC * _NS
_RPW = _B // _NW
_IMIN = np.int32(-2147483648)
_LO = np.float32(0.76)
_HI = np.float32(0.92)
_NEG_INF = np.float32(-np.inf)
_POS_INF = np.float32(np.inf)


def _keys(x):
    b = lax.bitcast_convert_type(x, jnp.int32)
    return jnp.where(b >= 0, b ^ _IMIN, ~b)


def _find_digit(hist_v, kprime, iota):
    S = hist_v[pl.ds(0, _L)]
    for j in range(1, 16):
        S = S + hist_v[pl.ds(j * _L, _L)]
    sh = jnp.flip(jnp.cumsum(jnp.flip(S)))
    cnt1 = jnp.sum((sh >= kprime).astype(jnp.int32))
    hstar = cnt1 - 1
    hab = jnp.sum(jnp.where(iota == cnt1, sh, 0))
    w = plsc.load_gather(hist_v, [iota * _L + hstar])
    sw = jnp.flip(jnp.cumsum(jnp.flip(w))) + hab
    cnt2 = jnp.sum((sw >= kprime).astype(jnp.int32))
    gtv = jnp.sum(jnp.where(iota == cnt2, sw, 0))
    gt = jnp.where(cnt2 == _L, hab, gtv)
    dstar = hstar * _L + cnt2 - 1
    return dstar, kprime - gt


def _compact(x_v, base, cand_v, lo_f, hi_f, zvec):
    def body(i, c):
        off, nhi_v = c
        x = x_v[pl.ds(base + i * _L, _L)]
        mhi = x >= hi_f
        match = (x >= lo_f) & jnp.logical_not(mhi)
        idx = off + jnp.cumsum(match.astype(jnp.int32)) - 1
        plsc.store_scatter(cand_v, [idx], x, mask=match)
        return (off + plsc.all_reduce_population_count(match),
                nhi_v + mhi.astype(jnp.int32))

    off, nhi_v = plsc.parallel_loop(0, _NV, 1, unroll=_U,
                                    carry=(zvec, zvec))(body)
    return jnp.max(off), jnp.sum(nhi_v)


def _select_row(x_v, base, cand_v, hist_v, iota, ones, zeros):
    k = jnp.int32(_K)
    m, nhi = _compact(x_v, base, cand_v, _LO, _HI, zeros)
    good = (nhi < k) & (k <= nhi + m)
    hi_side = nhi >= k
    lo_f = jnp.where(good, _LO, jnp.where(hi_side, _HI, _NEG_INF))
    hi_f = jnp.where(good, _HI, jnp.where(hi_side, _POS_INF, _LO))
    rank = jnp.where(good, k - nhi, jnp.where(hi_side, k, k - (nhi + m)))
    m_fin = jnp.where(good, m,
                      jnp.where(hi_side, nhi, jnp.int32(_D) - nhi - m))

    @pl.when(jnp.logical_not(good))
    def _():
        _compact(x_v, base, cand_v, lo_f, hi_f, zeros)

    plsc.store_scatter(cand_v, [jnp.broadcast_to(m_fin, (_L,)) + iota],
                       jnp.broadcast_to(hi_f, (_L,)))
    nblk = m_fin // _L + 1

    def radix_body(p, c):
        kprime, prefix, pmask = c
        shift = jnp.int32(24) - jnp.int32(8) * p
        for j in range(16):
            hist_v[pl.ds(j * _L, _L)] = zeros

        @plsc.parallel_loop(0, nblk, 1, unroll=2)
        def _(i):
            xc = cand_v[pl.ds(i * _L, _L)]
            ku = _keys(xc)
            inr = (xc >= lo_f) & jnp.logical_not(xc >= hi_f)
            match = inr & ((ku & pmask) == prefix)
            lo_nib = lax.shift_right_logical(ku, shift) & 0xF
            hi_nib = lax.shift_right_logical(ku, shift + 4) & 0xF
            tidx = lax.shift_left(lo_nib, 4) | hi_nib
            plsc.addupdate_scatter(hist_v, [tidx], ones, mask=match)

        d, kprime = _find_digit(hist_v, kprime, iota)
        prefix = prefix | lax.shift_left(d, shift)
        pmask = pmask | lax.shift_left(jnp.int32(0xFF), shift)
        return (kprime, prefix, pmask)

    kprime, prefix, pmask = lax.fori_loop(
        0, 4, radix_body, (rank, jnp.int32(0), jnp.int32(0)))

    bits = jnp.where(prefix < 0, prefix ^ _IMIN, ~prefix)
    return plsc.bitcast(jnp.broadcast_to(bits, (_L,)), jnp.float32)


def _mask_row(x_v, base, o_v, vstar_v):
    @plsc.parallel_loop(0, _NV, 1, unroll=_U)
    def _(i):
        x = x_v[pl.ds(base + i * _L, _L)]
        o_v[pl.ds(i * _L, _L)] = jnp.where(x >= vstar_v, x, 0.0)


def _kwta_body(x_hbm, o_hbm, x_v, o_v, cand_v, hist_v, sem0, sem1, semo):
    wid = lax.axis_index("s") * _NC + lax.axis_index("c")
    iota = lax.iota(jnp.int32, _L)
    ones = jnp.ones((_L,), jnp.int32)
    zeros = jnp.zeros((_L,), jnp.int32)
    row0 = wid * _RPW
    isems = [sem0, sem1]
    incopies = [pltpu.async_copy(x_hbm.at[row0 + r],
                                 x_v.at[pl.ds(r * _D, _D)], isems[r % 2])
                for r in range(_RPW)]
    for c in incopies:
        c.wait()

    def row_body(r, carry):
        base = r * jnp.int32(_D)
        vstar = _select_row(x_v, base, cand_v, hist_v, iota, ones, zeros)

        @pl.when(r > 0)
        def _():
            pltpu.make_async_copy(x_hbm.at[row0], o_v, semo).wait()
        _mask_row(x_v, base, o_v, vstar)
        pltpu.async_copy(o_v, o_hbm.at[row0 + r], semo)
        return carry

    lax.fori_loop(0, _RPW, row_body, jnp.int32(0))
    pltpu.make_async_copy(x_hbm.at[row0], o_v, semo).wait()


def kernel(inputs):
    mesh = plsc.VectorSubcoreMesh(
        core_axis_name="c", subcore_axis_name="s", num_cores=_NC,
        num_subcores=_NS)
    f = pl.kernel(
        _kwta_body,
        out_type=jax.ShapeDtypeStruct((_B, _D), jnp.float32),
        mesh=mesh,
        scratch_types=[
            pltpu.VMEM((_RPW * _D,), jnp.float32),
            pltpu.VMEM((_D,), jnp.float32),
            pltpu.VMEM((_D + _L,), jnp.float32),
            pltpu.VMEM((256,), jnp.int32),
            pltpu.SemaphoreType.DMA,
            pltpu.SemaphoreType.DMA,
            pltpu.SemaphoreType.DMA,
        ],
        compiler_params=pltpu.CompilerParams(needs_layout_passes=False),
    )
    return f(inputs)
